# Initial kernel scaffold; baseline (speedup 1.0000x reference)
import jax
import jax.numpy as jnp
from jax.experimental import pallas as pl


def kernel(x, sizes):
    raise NotImplementedError("write your pallas kernel here")


# speedup vs baseline: 3.1260x; 3.1260x over previous
import functools

import jax
import jax.numpy as jnp
from jax import lax
from jax.experimental import pallas as pl
from jax.experimental.pallas import tpu as pltpu
from jax.experimental.pallas import tpu_sc as plsc

B = 8
TMAX = 2048
D = 1024
NROWS = B * TMAX

NW = 32
C = 32


def _build_sc_unpack(total_rows: int):
    npad = NROWS - total_rows
    val_per_w = total_rows // NW
    pad_per_w = npad // NW
    val_chunks = val_per_w // C
    pad_chunks = pad_per_w // C

    info = plsc.get_sparse_core_info()
    nc = info.num_cores

    mesh = plsc.VectorSubcoreMesh(core_axis_name="c", subcore_axis_name="s")

    @functools.partial(
        pl.kernel,
        mesh=mesh,
        out_type=jax.ShapeDtypeStruct((NROWS, D), jnp.float32),
        scratch_types=[
            pltpu.VMEM((C,), jnp.int32),
            pltpu.VMEM((C, D), jnp.float32),
            pltpu.VMEM((C, D), jnp.float32),
            pltpu.SemaphoreType.DMA,
        ],
    )
    def unpack(x_hbm, dst_hbm, zdst_hbm, out_hbm, idx_v, rows_v, zero_v, sem):
        wid = lax.axis_index("s") * nc + lax.axis_index("c")

        zeros16 = jnp.zeros((16,), jnp.float32)
        lanes = D // 16

        def zbody(i, carry):
            r = i // lanes
            col = (i % lanes) * 16
            zero_v[r, pl.ds(col, 16)] = zeros16
            return carry

        lax.fori_loop(0, C * lanes, zbody, 0)

        for j in range(val_chunks):
            base = wid * val_per_w + j * C
            pltpu.sync_copy(x_hbm.at[pl.ds(base, C)], rows_v)
            pltpu.sync_copy(dst_hbm.at[pl.ds(base, C)], idx_v)
            pltpu.async_copy(rows_v, out_hbm.at[idx_v], sem).wait()

        for j in range(pad_chunks):
            zbase = wid * pad_per_w + j * C
            pltpu.sync_copy(zdst_hbm.at[pl.ds(zbase, C)], idx_v)
            pltpu.async_copy(zero_v, out_hbm.at[idx_v], sem).wait()

    return unpack, val_chunks, pad_chunks


def kernel(x, sizes):
    total_rows = x.shape[0]
    unpack, val_chunks, pad_chunks = _build_sc_unpack(total_rows)

    t = jnp.arange(TMAX, dtype=jnp.int32)
    sizes_i = sizes.astype(jnp.int32)
    bs = jnp.sum((sizes_i[None, :] > t[:, None]).astype(jnp.int32), axis=1)
    offsets = jnp.concatenate(
        [jnp.zeros((1,), jnp.int32), jnp.cumsum(bs)[:-1].astype(jnp.int32)]
    )
    valid = t[None, :] < sizes_i[:, None]
    src = offsets[None, :] + jnp.arange(B, dtype=jnp.int32)[:, None]
    vflat = valid.reshape(-1)
    sflat = src.reshape(-1)
    rows = jnp.arange(NROWS, dtype=jnp.int32)
    dst = (
        jnp.zeros((total_rows,), jnp.int32)
        .at[jnp.where(vflat, sflat, total_rows)]
        .set(rows, mode="drop")
    )
    zrows = jnp.nonzero(~vflat, size=NROWS - total_rows, fill_value=0)[0].astype(
        jnp.int32
    )

    out = unpack(x, dst, zrows)
    return (out.reshape(B, TMAX, D), sizes)


# speedup vs baseline: 3.4352x; 1.0989x over previous
import functools

import jax
import jax.numpy as jnp
from jax import lax
from jax.experimental import pallas as pl
from jax.experimental.pallas import tpu as pltpu
from jax.experimental.pallas import tpu_sc as plsc

B = 8
TMAX = 2048
D = 1024
NROWS = B * TMAX

NW = 32
C = 32


def _build_sc_unpack(total_rows: int):
    npad = NROWS - total_rows
    val_per_w = total_rows // NW
    pad_per_w = npad // NW
    val_chunks = val_per_w // C
    pad_chunks = pad_per_w // C
    nch = val_chunks + pad_chunks

    info = plsc.get_sparse_core_info()
    nc = info.num_cores

    mesh = plsc.VectorSubcoreMesh(core_axis_name="c", subcore_axis_name="s")

    @functools.partial(
        pl.kernel,
        mesh=mesh,
        out_type=jax.ShapeDtypeStruct((NROWS, D), jnp.float32),
        scratch_types=[
            pltpu.VMEM((C,), jnp.int32),
            pltpu.VMEM((C,), jnp.int32),
            pltpu.VMEM((C, D), jnp.float32),
            pltpu.VMEM((C, D), jnp.float32),
            pltpu.VMEM((C, D), jnp.float32),
            pltpu.SemaphoreType.DMA,
            pltpu.SemaphoreType.DMA,
        ],
    )
    def unpack(x_hbm, dst_hbm, zdst_hbm, zsrc_hbm, out_hbm,
               idx0, idx1, rows0, rows1, zero_v, lsem, ssem):
        wid = lax.axis_index("s") * nc + lax.axis_index("c")
        rows = (rows0, rows1)
        idxb = (idx0, idx1)

        def start_load(j):
            cps = []
            if j < val_chunks:
                base = wid * val_per_w + j * C
                cps.append(pltpu.async_copy(x_hbm.at[pl.ds(base, C)], rows[j % 2], lsem))
                cps.append(pltpu.async_copy(dst_hbm.at[pl.ds(base, C)], idxb[j % 2], lsem))
            else:
                zbase = wid * pad_per_w + (j - val_chunks) * C
                cps.append(pltpu.async_copy(zdst_hbm.at[pl.ds(zbase, C)], idxb[j % 2], lsem))
            return cps

        zload = pltpu.async_copy(zsrc_hbm, zero_v, lsem)
        loads = {0: start_load(0)}
        scats = {}
        zload.wait()
        for j in range(nch):
            for cp in loads[j]:
                cp.wait()
            src = rows[j % 2] if j < val_chunks else zero_v
            scats[j] = pltpu.async_copy(src, out_hbm.at[idxb[j % 2]], ssem)
            if j >= 1:
                scats[j - 1].wait()
            if j + 1 < nch:
                loads[j + 1] = start_load(j + 1)
        scats[nch - 1].wait()

    return unpack, val_chunks, pad_chunks


def kernel(x, sizes):
    total_rows = x.shape[0]
    unpack, val_chunks, pad_chunks = _build_sc_unpack(total_rows)

    t = jnp.arange(TMAX, dtype=jnp.int32)
    sizes_i = sizes.astype(jnp.int32)
    bs = jnp.sum((sizes_i[None, :] > t[:, None]).astype(jnp.int32), axis=1)
    offsets = jnp.concatenate(
        [jnp.zeros((1,), jnp.int32), jnp.cumsum(bs)[:-1].astype(jnp.int32)]
    )
    valid = t[None, :] < sizes_i[:, None]
    src = offsets[None, :] + jnp.arange(B, dtype=jnp.int32)[:, None]
    vflat = valid.reshape(-1)
    sflat = src.reshape(-1)
    rows = jnp.arange(NROWS, dtype=jnp.int32)
    dst = (
        jnp.zeros((total_rows,), jnp.int32)
        .at[jnp.where(vflat, sflat, total_rows)]
        .set(rows, mode="drop")
    )
    zrows = jnp.nonzero(~vflat, size=NROWS - total_rows, fill_value=0)[0].astype(
        jnp.int32
    )
    zsrc = jnp.zeros((C, D), jnp.float32)

    out = unpack(x, dst, zrows, zsrc)
    return (out.reshape(B, TMAX, D), sizes)


# speedup vs baseline: 5.3093x; 1.5456x over previous
import functools

import jax
import jax.numpy as jnp
from jax import lax
from jax.experimental import pallas as pl
from jax.experimental.pallas import tpu as pltpu
from jax.experimental.pallas import tpu_sc as plsc

B = 8
TMAX = 2048
D = 1024
NROWS = B * TMAX

NW = 32
C = 32


def _build_sc_unpack(total_rows: int):
    npad = NROWS - total_rows
    val_per_w = total_rows // NW
    pad_per_w = npad // NW
    val_chunks = val_per_w // C
    pad_chunks = pad_per_w // C
    nch = val_chunks + pad_chunks

    info = plsc.get_sparse_core_info()
    nc = info.num_cores

    mesh = plsc.VectorSubcoreMesh(core_axis_name="c", subcore_axis_name="s")

    @functools.partial(
        pl.kernel,
        mesh=mesh,
        out_type=jax.ShapeDtypeStruct((NROWS, D), jnp.float32),
        scratch_types=[
            pltpu.VMEM((C,), jnp.int32),
            pltpu.VMEM((C,), jnp.int32),
            pltpu.VMEM((C, D), jnp.float32),
            pltpu.VMEM((C, D), jnp.float32),
            pltpu.VMEM((C, D), jnp.float32),
            pltpu.SemaphoreType.DMA,
            pltpu.SemaphoreType.DMA,
        ],
    )
    def unpack(x_hbm, dst_hbm, zdst_hbm, zsrc_hbm, out_hbm,
               idx0, idx1, rows0, rows1, zero_v, lsem, ssem):
        wid = lax.axis_index("s") * nc + lax.axis_index("c")
        rows = (rows0, rows1)
        idxb = (idx0, idx1)

        def start_load(j):
            cps = []
            if j < val_chunks:
                base = wid * val_per_w + j * C
                cps.append(pltpu.async_copy(x_hbm.at[pl.ds(base, C)], rows[j % 2], lsem))
                cps.append(pltpu.async_copy(dst_hbm.at[pl.ds(base, C)], idxb[j % 2], lsem))
            else:
                zbase = wid * pad_per_w + (j - val_chunks) * C
                cps.append(pltpu.async_copy(zdst_hbm.at[pl.ds(zbase, C)], idxb[j % 2], lsem))
            return cps

        zload = pltpu.async_copy(zsrc_hbm, zero_v, lsem)
        loads = {0: start_load(0)}
        scats = {}
        zload.wait()
        for j in range(nch):
            for cp in loads[j]:
                cp.wait()
            src = rows[j % 2] if j < val_chunks else zero_v
            scats[j] = pltpu.async_copy(src, out_hbm.at[idxb[j % 2]], ssem)
            if j >= 1:
                scats[j - 1].wait()
            if j + 1 < nch:
                loads[j + 1] = start_load(j + 1)
        scats[nch - 1].wait()

    return unpack, val_chunks, pad_chunks


def kernel(x, sizes):
    total_rows = x.shape[0]
    unpack, val_chunks, pad_chunks = _build_sc_unpack(total_rows)

    t = jnp.arange(TMAX, dtype=jnp.int32)
    sizes_i = sizes.astype(jnp.int32)
    bs = jnp.sum((sizes_i[None, :] > t[:, None]).astype(jnp.int32), axis=1)
    offsets = jnp.concatenate(
        [jnp.zeros((1,), jnp.int32), jnp.cumsum(bs)[:-1].astype(jnp.int32)]
    )
    p = jnp.arange(total_rows, dtype=jnp.int32)
    cmp = offsets[None, :] <= p[:, None]
    t_p = jnp.sum(cmp.astype(jnp.int32), axis=1) - 1
    off_p = jnp.max(jnp.where(cmp, offsets[None, :], 0), axis=1)
    dst = (p - off_p) * TMAX + t_p
    padlen = TMAX - sizes_i
    pcum = jnp.concatenate(
        [jnp.zeros((1,), jnp.int32), jnp.cumsum(padlen)[:-1].astype(jnp.int32)]
    )
    k = jnp.arange(NROWS - total_rows, dtype=jnp.int32)
    cmpz = pcum[None, :] <= k[:, None]
    b_k = jnp.sum(cmpz.astype(jnp.int32), axis=1) - 1
    pc_k = jnp.max(jnp.where(cmpz, pcum[None, :], 0), axis=1)
    sz_k = jnp.min(jnp.where(cmpz, sizes_i[None, :], TMAX), axis=1)
    zrows = b_k * TMAX + sz_k + (k - pc_k)
    zsrc = jnp.zeros((C, D), jnp.float32)

    out = unpack(x, dst, zrows, zsrc)
    return (out.reshape(B, TMAX, D), sizes)


# speedup vs baseline: 6.9078x; 1.3011x over previous
import functools

import jax
import jax.numpy as jnp
from jax import lax
from jax.experimental import pallas as pl
from jax.experimental.pallas import tpu as pltpu
from jax.experimental.pallas import tpu_sc as plsc

B = 8
TMAX = 2048
D = 1024
NROWS = B * TMAX

NW = 32
C = 32
L = 16

_T_SIZES = 0
_T_E = 16
_T_O = 32
_T_PCUM = 48


def _build_sc_unpack(total_rows: int):
    npad = NROWS - total_rows
    val_per_w = total_rows // NW
    pad_per_w = npad // NW
    val_chunks = val_per_w // C
    pad_chunks = pad_per_w // C
    nch = val_chunks + pad_chunks

    info = plsc.get_sparse_core_info()
    nc = info.num_cores

    mesh = plsc.VectorSubcoreMesh(core_axis_name="c", subcore_axis_name="s")

    scratch = [pltpu.VMEM((C,), jnp.int32) for _ in range(nch)]
    scratch += [
        pltpu.VMEM((C, D), jnp.float32),
        pltpu.VMEM((C, D), jnp.float32),
        pltpu.VMEM((C, D), jnp.float32),
        pltpu.VMEM((L,), jnp.int32),
        pltpu.SemaphoreType.DMA,
        pltpu.SemaphoreType.DMA,
    ]

    @functools.partial(
        pl.kernel,
        mesh=mesh,
        out_type=jax.ShapeDtypeStruct((NROWS, D), jnp.float32),
        scratch_types=scratch,
    )
    def unpack(x_hbm, sizes_hbm, out_hbm, *refs):
        idxs = refs[:nch]
        rows0, rows1, zero_v, tbl, lsem, ssem = refs[nch:]
        rows = (rows0, rows1)
        wid = lax.axis_index("s") * nc + lax.axis_index("c")

        def start_load(j):
            base = wid * val_per_w + j * C
            return pltpu.async_copy(x_hbm.at[pl.ds(base, C)], rows[j % 2], lsem)

        loads = {0: start_load(0), 1: start_load(1)}

        pltpu.sync_copy(sizes_hbm, tbl)
        iota = jax.lax.broadcasted_iota(jnp.int32, (L,), 0)
        zeros_i = jnp.zeros((L,), jnp.int32)

        sizes_v = tbl[...]
        sz = [sizes_v[b] for b in range(B)]
        e_t = [0] + [sz[B - r] for r in range(1, B + 1)]
        o_t = [sum((jnp.minimum(sz[b], e_t[r]) for b in range(B)), 0)
               for r in range(B + 1)]
        pc_t = [sum(((TMAX - sz[bp]) for bp in range(b)), 0) for b in range(B)]

        def vsplat(s):
            return jnp.broadcast_to(jnp.asarray(s, jnp.int32), (L,))

        o_spl = [vsplat(o_t[r]) for r in range(1, B + 1)]
        e_spl = [vsplat(e_t[r]) for r in range(1, B + 1)]
        pc_spl = [vsplat(pc_t[b]) for b in range(1, B)]
        sz_spl = [vsplat(sz[b]) for b in range(B)]

        ones_i = jnp.ones((L,), jnp.int32)

        for j in range(val_chunks):
            for h in range(C // L):
                pvec = (wid * val_per_w + j * C + h * L) + iota
                r_p = zeros_i
                o_sel = zeros_i
                e_sel = zeros_i
                for r in range(B):
                    ge = pvec >= o_spl[r]
                    r_p = r_p + jnp.where(ge, ones_i, zeros_i)
                    o_sel = jnp.where(ge, o_spl[r], o_sel)
                    e_sel = jnp.where(ge, e_spl[r], e_sel)
                bs_p = B - r_p
                rel = pvec - o_sel
                dstv = lax.rem(rel, bs_p) * TMAX + e_sel + lax.div(rel, bs_p)
                idxs[j][pl.ds(h * L, L)] = dstv

        for j in range(pad_chunks):
            for h in range(C // L):
                kvec = (wid * pad_per_w + j * C + h * L) + iota
                b_k = zeros_i
                pc_sel = zeros_i
                sz_sel = sz_spl[0]
                for b in range(1, B):
                    ge = kvec >= pc_spl[b - 1]
                    b_k = b_k + jnp.where(ge, ones_i, zeros_i)
                    pc_sel = jnp.where(ge, pc_spl[b - 1], pc_sel)
                    sz_sel = jnp.where(ge, sz_spl[b], sz_sel)
                zrv = b_k * TMAX + sz_sel + (kvec - pc_sel)
                idxs[val_chunks + j][pl.ds(h * L, L)] = zrv

        zeros_f = jnp.zeros((L,), jnp.float32)

        def zfill(r, carry):
            for cidx in range(D // L):
                zero_v[r, pl.ds(cidx * L, L)] = zeros_f
            return carry

        lax.fori_loop(0, C, zfill, 0)

        scats = {}
        for j in range(nch):
            if j < val_chunks:
                loads[j].wait()
                src = rows[j % 2]
            else:
                src = zero_v
            scats[j] = pltpu.async_copy(src, out_hbm.at[idxs[j]], ssem)
            nxt = j + 2
            if nxt < val_chunks:
                scats[j].wait()
                loads[nxt] = start_load(nxt)
        for j in range(max(0, val_chunks - 2), nch):
            scats[j].wait()

    return unpack


def kernel(x, sizes):
    total_rows = x.shape[0]
    unpack = _build_sc_unpack(total_rows)
    sizes16 = jnp.concatenate(
        [sizes.astype(jnp.int32), jnp.zeros((16 - B,), jnp.int32)]
    )
    out = unpack(x, sizes16)
    return (out.reshape(B, TMAX, D), sizes)


# speedup vs baseline: 7.1199x; 1.0307x over previous
import functools

import jax
import jax.numpy as jnp
from jax import lax
from jax.experimental import pallas as pl
from jax.experimental.pallas import tpu as pltpu
from jax.experimental.pallas import tpu_sc as plsc

B = 8
TMAX = 2048
D = 1024
NROWS = B * TMAX

NW = 32
CV = 48
CZ = 16
L = 16


def _build_sc_unpack(total_rows: int):
    npad = NROWS - total_rows
    val_per_w = total_rows // NW
    pad_per_w = npad // NW
    val_chunks = val_per_w // CV
    pad_chunks = pad_per_w // CZ
    nch = val_chunks + pad_chunks

    info = plsc.get_sparse_core_info()
    nc = info.num_cores

    mesh = plsc.VectorSubcoreMesh(core_axis_name="c", subcore_axis_name="s")

    scratch = [pltpu.VMEM((CV,), jnp.int32) for _ in range(val_chunks)]
    scratch += [pltpu.VMEM((CZ,), jnp.int32) for _ in range(pad_chunks)]
    scratch += [
        pltpu.VMEM((CV, D), jnp.float32),
        pltpu.VMEM((CV, D), jnp.float32),
        pltpu.VMEM((CZ, D), jnp.float32),
        pltpu.VMEM((L,), jnp.int32),
        pltpu.SemaphoreType.DMA,
        pltpu.SemaphoreType.DMA,
    ]

    @functools.partial(
        pl.kernel,
        mesh=mesh,
        out_type=jax.ShapeDtypeStruct((NROWS, D), jnp.float32),
        scratch_types=scratch,
    )
    def unpack(x_hbm, sizes_hbm, out_hbm, *refs):
        idxs = refs[:nch]
        rows0, rows1, zero_v, tbl, lsem, ssem = refs[nch:]
        rows = (rows0, rows1)
        wid = lax.axis_index("s") * nc + lax.axis_index("c")

        def start_load(j):
            base = wid * val_per_w + j * CV
            return pltpu.async_copy(x_hbm.at[pl.ds(base, CV)], rows[j % 2], lsem)

        loads = {0: start_load(0), 1: start_load(1)}

        pltpu.sync_copy(sizes_hbm, tbl)
        iota = jax.lax.broadcasted_iota(jnp.int32, (L,), 0)
        zeros_i = jnp.zeros((L,), jnp.int32)

        sizes_v = tbl[...]
        sz = [sizes_v[b] for b in range(B)]
        e_t = [0] + [sz[B - r] for r in range(1, B + 1)]
        o_t = [sum((jnp.minimum(sz[b], e_t[r]) for b in range(B)), 0)
               for r in range(B + 1)]
        pc_t = [sum(((TMAX - sz[bp]) for bp in range(b)), 0) for b in range(B)]

        def vsplat(s):
            return jnp.broadcast_to(jnp.asarray(s, jnp.int32), (L,))

        o_spl = [vsplat(o_t[r]) for r in range(1, B + 1)]
        e_spl = [vsplat(e_t[r]) for r in range(1, B + 1)]
        pc_spl = [vsplat(pc_t[b]) for b in range(1, B)]
        sz_spl = [vsplat(sz[b]) for b in range(B)]

        ones_i = jnp.ones((L,), jnp.int32)

        for j in range(val_chunks):
            for h in range(CV // L):
                pvec = (wid * val_per_w + j * CV + h * L) + iota
                r_p = zeros_i
                o_sel = zeros_i
                e_sel = zeros_i
                for r in range(B):
                    ge = pvec >= o_spl[r]
                    r_p = r_p + jnp.where(ge, ones_i, zeros_i)
                    o_sel = jnp.where(ge, o_spl[r], o_sel)
                    e_sel = jnp.where(ge, e_spl[r], e_sel)
                bs_p = B - r_p
                rel = pvec - o_sel
                dstv = lax.rem(rel, bs_p) * TMAX + e_sel + lax.div(rel, bs_p)
                idxs[j][pl.ds(h * L, L)] = dstv

        for j in range(pad_chunks):
            for h in range(CZ // L):
                kvec = (wid * pad_per_w + j * CZ + h * L) + iota
                b_k = zeros_i
                pc_sel = zeros_i
                sz_sel = sz_spl[0]
                for b in range(1, B):
                    ge = kvec >= pc_spl[b - 1]
                    b_k = b_k + jnp.where(ge, ones_i, zeros_i)
                    pc_sel = jnp.where(ge, pc_spl[b - 1], pc_sel)
                    sz_sel = jnp.where(ge, sz_spl[b], sz_sel)
                zrv = b_k * TMAX + sz_sel + (kvec - pc_sel)
                idxs[val_chunks + j][pl.ds(h * L, L)] = zrv

        zeros_f = jnp.zeros((L,), jnp.float32)

        def zfill(r, carry):
            for cidx in range(D // L):
                zero_v[r, pl.ds(cidx * L, L)] = zeros_f
            return carry

        lax.fori_loop(0, CZ, zfill, 0)

        scats = {}
        for j in range(nch):
            if j < val_chunks:
                loads[j].wait()
                src = rows[j % 2]
            else:
                src = zero_v
            scats[j] = pltpu.async_copy(src, out_hbm.at[idxs[j]], ssem)
            nxt = j + 2
            if nxt < val_chunks:
                scats[j].wait()
                loads[nxt] = start_load(nxt)
        for j in range(max(0, val_chunks - 2), nch):
            scats[j].wait()

    return unpack


def kernel(x, sizes):
    total_rows = x.shape[0]
    unpack = _build_sc_unpack(total_rows)
    sizes16 = jnp.concatenate(
        [sizes.astype(jnp.int32), jnp.zeros((16 - B,), jnp.int32)]
    )
    out = unpack(x, sizes16)
    return (out.reshape(B, TMAX, D), sizes)
